# baseline (speedup 1.0000x reference)
import functools

import jax
import jax.numpy as jnp
from jax import lax
from jax.experimental import pallas as pl
from jax.experimental.pallas import tpu as pltpu
from jax.experimental.pallas import tpu_sc as plsc

B = 16384
H = 50
D = 64
L = 16
NDC = D // L
REG_COEF = 0.01

_info = plsc.get_sparse_core_info()
NC, NS = _info.num_cores, _info.num_subcores
NW = NC * NS
BPW = B // NW
CB = 8
NSUB = BPW // CB
HB = CB * H

_mesh = plsc.VectorSubcoreMesh(core_axis_name="c", subcore_axis_name="s")


@functools.partial(
    pl.kernel,
    mesh=_mesh,
    compiler_params=pltpu.CompilerParams(
        needs_layout_passes=False, use_tc_tiling_on_sc=False),
    out_type=jax.ShapeDtypeStruct((B + NW * L,), jnp.float32),
    scratch_types=[
        pltpu.VMEM((BPW,), jnp.int32),
        pltpu.VMEM((BPW * H,), jnp.int32),
        pltpu.VMEM((BPW * D,), jnp.float32),
        pltpu.VMEM((BPW * L,), jnp.float32),
        pltpu.VMEM((CB, D), jnp.float32),
        pltpu.VMEM((CB, D), jnp.float32),
        pltpu.VMEM((HB, D), jnp.float32),
        pltpu.VMEM((HB, D), jnp.float32),
        pltpu.VMEM((BPW,), jnp.float32),
        pltpu.VMEM((L,), jnp.float32),
        pltpu.SemaphoreType.DMA,
        pltpu.SemaphoreType.DMA,
        pltpu.SemaphoreType.DMA,
        pltpu.SemaphoreType.DMA,
    ],
)
def _cf_kernel(idx_all, Wu, Wi, Wur, Wir,
               out,
               idx_v, hidx_v, urepr_v, dotp_v, row0_v, row1_v,
               hrow0_v, hrow1_v, dot_v, reg_v,
               sem_r0, sem_r1, sem_h0, sem_h1):
    wid = lax.axis_index("s") * NC + lax.axis_index("c")
    base = wid * BPW

    reg_v[...] = jnp.zeros((L,), jnp.float32)

    row_bufs = (row0_v, row1_v)
    hrow_bufs = (hrow0_v, hrow1_v)
    row_sems = (sem_r0, sem_r1)
    hrow_sems = (sem_h0, sem_h1)

    def run_pass(ids_off, hist_off, tbl_hbm, htbl_hbm, is_user_pass):
        pltpu.sync_copy(idx_all.at[pl.ds(ids_off + base, BPW)], idx_v)
        pltpu.sync_copy(
            idx_all.at[pl.ds(hist_off + base * H, BPW * H)], hidx_v)

        def start(c, k):
            pltpu.async_copy(
                tbl_hbm.at[idx_v.at[pl.ds(c * CB, CB)]],
                row_bufs[k], row_sems[k])
            pltpu.async_copy(
                htbl_hbm.at[hidx_v.at[pl.ds(c * HB, HB)]],
                hrow_bufs[k], hrow_sems[k])

        def wait(c, k):
            pltpu.make_async_copy(
                tbl_hbm.at[idx_v.at[pl.ds(c * CB, CB)]],
                row_bufs[k], row_sems[k]).wait()
            pltpu.make_async_copy(
                htbl_hbm.at[hidx_v.at[pl.ds(c * HB, HB)]],
                hrow_bufs[k], hrow_sems[k]).wait()

        start(0, 0)

        def pair(t, carry):
            for k in range(2):
                c = 2 * t + k

                @pl.when(c + 1 < NSUB)
                def _():
                    start(c + 1, 1 - k)

                wait(c, k)
                row_v = row_bufs[k]
                hrow_v = hrow_bufs[k]
                for r in range(CB):
                    rg = c * CB + r

                    def hbody(h, accs):
                        return tuple(
                            accs[d] + hrow_v[r * H + h, pl.ds(L * d, L)]
                            for d in range(NDC))

                    accs = lax.fori_loop(
                        0, H, hbody,
                        tuple(jnp.zeros((L,), jnp.float32)
                              for _ in range(NDC)),
                        unroll=25)
                    uv = [row_v[r, pl.ds(L * d, L)] for d in range(NDC)]
                    s = accs[0] * accs[0] + uv[0] * uv[0]
                    for d in range(1, NDC):
                        s = s + accs[d] * accs[d] + uv[d] * uv[d]
                    reg_v[...] = reg_v[...] + s
                    if is_user_pass:
                        for d in range(NDC):
                            urepr_v[pl.ds(rg * D + L * d, L)] = \
                                uv[d] + accs[d]
                    else:
                        tmp = ((uv[0] + accs[0])
                               * urepr_v[pl.ds(rg * D, L)])
                        for d in range(1, NDC):
                            tmp = tmp + ((uv[d] + accs[d])
                                         * urepr_v[pl.ds(rg * D + L * d, L)])
                        dotp_v[pl.ds(rg * L, L)] = tmp
            return carry

        lax.fori_loop(0, NSUB // 2, pair, 0)

    run_pass(0, 2 * B, Wu, Wur, True)
    run_pass(B, 2 * B + B * H, Wi, Wir, False)

    def dot_group(g, carry):
        rows = g * L + lax.iota(jnp.int32, L)
        acc = jnp.zeros((L,), jnp.float32)
        for l in range(L):
            acc = acc + plsc.load_gather(dotp_v, [rows * L + l])
        dot_v[pl.ds(g * L, L)] = acc
        return carry

    lax.fori_loop(0, BPW // L, dot_group, 0)

    pltpu.sync_copy(dot_v, out.at[pl.ds(base, BPW)])
    pltpu.sync_copy(reg_v, out.at[pl.ds(B + wid * L, L)])


def kernel(users, items, users_sparse_ratings, items_sparse_ratings,
           Wu, Wi, Wur, Wir):
    idx_all = jnp.concatenate([
        users, items,
        users_sparse_ratings.reshape(-1),
        items_sparse_ratings.reshape(-1),
    ])
    out = _cf_kernel(idx_all, Wu, Wi, Wur, Wir)
    return out[:B].reshape(B, 1), REG_COEF * jnp.sum(out[B:])


# speedup vs baseline: 2.4147x; 2.4147x over previous
import functools

import jax
import jax.numpy as jnp
from jax import lax
from jax.experimental import pallas as pl
from jax.experimental.pallas import tpu as pltpu
from jax.experimental.pallas import tpu_sc as plsc

B = 16384
H = 50
D = 64
L = 16
NDC = D // L
REG_COEF = 0.01

_info = plsc.get_sparse_core_info()
NC, NS = _info.num_cores, _info.num_subcores
NW = NC * NS
BPW = B // NW
CB = 8
NSUB = BPW // CB
HB = CB * H

_mesh = plsc.VectorSubcoreMesh(core_axis_name="c", subcore_axis_name="s")


@functools.partial(
    pl.kernel,
    mesh=_mesh,
    compiler_params=pltpu.CompilerParams(
        needs_layout_passes=False, use_tc_tiling_on_sc=False),
    out_type=jax.ShapeDtypeStruct((B + NW * L,), jnp.float32),
    scratch_types=[
        pltpu.VMEM((BPW,), jnp.float32),
        pltpu.VMEM((BPW,), jnp.int32),
        pltpu.VMEM((BPW * H,), jnp.int32),
        pltpu.VMEM((BPW * D,), jnp.float32),
        pltpu.VMEM((BPW * L,), jnp.float32),
        pltpu.VMEM((CB, D), jnp.float32),
        pltpu.VMEM((CB, D), jnp.float32),
        pltpu.VMEM((HB, D), jnp.float32),
        pltpu.VMEM((HB, D), jnp.float32),
        pltpu.VMEM((BPW,), jnp.float32),
        pltpu.VMEM((L,), jnp.float32),
        pltpu.SemaphoreType.DMA,
        pltpu.SemaphoreType.DMA,
        pltpu.SemaphoreType.DMA,
        pltpu.SemaphoreType.DMA,
    ],
)
def _cf_kernel(users, items, usr, isr, Wu, Wi, Wur, Wir,
               out,
               idxf_v, idx_v, hidx_v, urepr_v, dotp_v, row0_v, row1_v,
               hrow0_v, hrow1_v, dot_v, reg_v,
               sem_r0, sem_r1, sem_h0, sem_h1):
    wid = lax.axis_index("s") * NC + lax.axis_index("c")
    base = wid * BPW

    reg_v[...] = jnp.zeros((L,), jnp.float32)

    row_bufs = (row0_v, row1_v)
    hrow_bufs = (hrow0_v, hrow1_v)
    row_sems = (sem_r0, sem_r1)
    hrow_sems = (sem_h0, sem_h1)

    def run_pass(ids_hbm, hist_hbm, tbl_hbm, htbl_hbm, is_user_pass):
        pltpu.sync_copy(ids_hbm.at[pl.ds(base, BPW)], idxf_v)
        for b in range(BPW // L):
            idx_v[pl.ds(b * L, L)] = plsc.bitcast(
                idxf_v[pl.ds(b * L, L)], jnp.int32)
        pltpu.sync_copy(hist_hbm.at[pl.ds(base * H, BPW * H)], hidx_v)

        def start(c, k):
            pltpu.async_copy(
                tbl_hbm.at[idx_v.at[pl.ds(c * CB, CB)]],
                row_bufs[k], row_sems[k])
            pltpu.async_copy(
                htbl_hbm.at[hidx_v.at[pl.ds(c * HB, HB)]],
                hrow_bufs[k], hrow_sems[k])

        def wait(c, k):
            pltpu.make_async_copy(
                tbl_hbm.at[idx_v.at[pl.ds(c * CB, CB)]],
                row_bufs[k], row_sems[k]).wait()
            pltpu.make_async_copy(
                htbl_hbm.at[hidx_v.at[pl.ds(c * HB, HB)]],
                hrow_bufs[k], hrow_sems[k]).wait()

        start(0, 0)

        def pair(t, carry):
            for k in range(2):
                c = 2 * t + k

                @pl.when(c + 1 < NSUB)
                def _():
                    start(c + 1, 1 - k)

                wait(c, k)
                row_v = row_bufs[k]
                hrow_v = hrow_bufs[k]
                for r in range(CB):
                    rg = c * CB + r

                    def hbody(h, accs):
                        return tuple(
                            accs[d] + hrow_v[r * H + h, pl.ds(L * d, L)]
                            for d in range(NDC))

                    accs = lax.fori_loop(
                        0, H, hbody,
                        tuple(jnp.zeros((L,), jnp.float32)
                              for _ in range(NDC)),
                        unroll=25)
                    uv = [row_v[r, pl.ds(L * d, L)] for d in range(NDC)]
                    s = accs[0] * accs[0] + uv[0] * uv[0]
                    for d in range(1, NDC):
                        s = s + accs[d] * accs[d] + uv[d] * uv[d]
                    reg_v[...] = reg_v[...] + s
                    if is_user_pass:
                        for d in range(NDC):
                            urepr_v[pl.ds(rg * D + L * d, L)] = \
                                uv[d] + accs[d]
                    else:
                        tmp = ((uv[0] + accs[0])
                               * urepr_v[pl.ds(rg * D, L)])
                        for d in range(1, NDC):
                            tmp = tmp + ((uv[d] + accs[d])
                                         * urepr_v[pl.ds(rg * D + L * d, L)])
                        dotp_v[pl.ds(rg * L, L)] = tmp
            return carry

        lax.fori_loop(0, NSUB // 2, pair, 0)

    run_pass(users, usr, Wu, Wur, True)
    run_pass(items, isr, Wi, Wir, False)

    def dot_group(g, carry):
        rows = g * L + lax.iota(jnp.int32, L)
        acc = jnp.zeros((L,), jnp.float32)
        for l in range(L):
            acc = acc + plsc.load_gather(dotp_v, [rows * L + l])
        dot_v[pl.ds(g * L, L)] = acc
        return carry

    lax.fori_loop(0, BPW // L, dot_group, 0)

    pltpu.sync_copy(dot_v, out.at[pl.ds(base, BPW)])
    pltpu.sync_copy(reg_v, out.at[pl.ds(B + wid * L, L)])


def kernel(users, items, users_sparse_ratings, items_sparse_ratings,
           Wu, Wi, Wur, Wir):
    uf = lax.bitcast_convert_type(users, jnp.float32)
    itf = lax.bitcast_convert_type(items, jnp.float32)
    out = _cf_kernel(uf, itf, users_sparse_ratings.reshape(-1),
                     items_sparse_ratings.reshape(-1), Wu, Wi, Wur, Wir)
    return out[:B].reshape(B, 1), REG_COEF * jnp.sum(out[B:])
